# baseline (speedup 1.0000x reference)
import jax
import jax.numpy as jnp
from jax.experimental import pallas as pl
from jax.experimental.pallas import tpu as pltpu


def _wnn_kernel(x_ref, p_ref, w1_ref, b1_ref, w2_ref, b2_ref, ens_ref, w_ref):
    x = x_ref[...]
    h = jnp.dot(x, w1_ref[...], preferred_element_type=jnp.float32)
    h = jnp.maximum(h + b1_ref[...], 0.0)
    raw = jnp.dot(h, w2_ref[...], preferred_element_type=jnp.float32)
    w = jax.nn.sigmoid(raw + b2_ref[...])
    w_ref[...] = w
    ens_ref[...] = jnp.sum(w * p_ref[...], axis=1, keepdims=True)


def kernel(x_nn_scaled, target, pred_HBV, pred_PRMS, pred_SACSMA, W1, b1, W2, b2):
    T, B, NX = x_nn_scaled.shape
    Tt = target.shape[0]
    H = W1.shape[1]
    M = W2.shape[1]
    diff = T - Tt
    N = Tt * B
    OFF = diff * B

    C = 9344
    assert N % C == 0 and OFF % C == 0
    grid = N // C
    off_blocks = OFF // C

    x2 = x_nn_scaled.reshape(T * B, NX)
    preds = jnp.concatenate(
        [pred_HBV.reshape(N, 1), pred_PRMS.reshape(N, 1), pred_SACSMA.reshape(N, 1)],
        axis=1,
    )
    b1r = b1.reshape(1, H)
    b2r = b2.reshape(1, M)

    ens, w = pl.pallas_call(
        _wnn_kernel,
        grid=(grid,),
        in_specs=[
            pl.BlockSpec((C, NX), lambda i: (i + off_blocks, 0)),
            pl.BlockSpec((C, M), lambda i: (i, 0)),
            pl.BlockSpec((NX, H), lambda i: (0, 0)),
            pl.BlockSpec((1, H), lambda i: (0, 0)),
            pl.BlockSpec((H, M), lambda i: (0, 0)),
            pl.BlockSpec((1, M), lambda i: (0, 0)),
        ],
        out_specs=[
            pl.BlockSpec((C, 1), lambda i: (i, 0)),
            pl.BlockSpec((C, M), lambda i: (i, 0)),
        ],
        out_shape=[
            jax.ShapeDtypeStruct((N, 1), jnp.float32),
            jax.ShapeDtypeStruct((N, M), jnp.float32),
        ],
        compiler_params=pltpu.CompilerParams(
            dimension_semantics=("arbitrary",),
        ),
    )(x2, preds, W1, b1r, W2, b2r)

    return ens.reshape(Tt, B), w.reshape(Tt, B, M)


# speedup vs baseline: 2.5511x; 2.5511x over previous
import jax
import jax.numpy as jnp
from jax.experimental import pallas as pl
from jax.experimental.pallas import tpu as pltpu


def _wnn_kernel(x_ref, p_ref, w1_ref, b1_ref, w2_ref, b2_ref, w_ref, ens_ref):
    xt = x_ref[...]
    h = jnp.dot(w1_ref[...], xt, preferred_element_type=jnp.float32)
    h = jnp.maximum(h + b1_ref[...], 0.0)
    raw = jnp.dot(w2_ref[...], h.astype(jnp.bfloat16),
                  preferred_element_type=jnp.float32)
    w8 = jax.nn.sigmoid(raw + b2_ref[...])
    w_ref[...] = w8
    ens_ref[...] = jnp.sum(w8 * p_ref[...], axis=0, keepdims=True)


def kernel(x_nn_scaled, target, pred_HBV, pred_PRMS, pred_SACSMA, W1, b1, W2, b2):
    T, B, NX = x_nn_scaled.shape
    Tt = target.shape[0]
    H = W1.shape[1]
    M = W2.shape[1]
    diff = T - Tt
    N = Tt * B
    OFF = diff * B

    C = 9344
    assert N % C == 0 and OFF % C == 0
    grid = N // C
    off_blocks = OFF // C

    MP = 8

    xT = x_nn_scaled.reshape(T * B, NX).T.astype(jnp.bfloat16)
    predsT = jnp.concatenate(
        [
            pred_HBV.reshape(1, N),
            pred_PRMS.reshape(1, N),
            pred_SACSMA.reshape(1, N),
            jnp.zeros((MP - M, N), jnp.float32),
        ],
        axis=0,
    )
    w1T = W1.T.astype(jnp.bfloat16)
    b1c = b1.reshape(H, 1)
    w2T = jnp.zeros((MP, H), jnp.bfloat16).at[:M].set(W2.T.astype(jnp.bfloat16))
    b2c = jnp.zeros((MP, 1), jnp.float32).at[:M, 0].set(b2)

    w8, ens = pl.pallas_call(
        _wnn_kernel,
        grid=(grid,),
        in_specs=[
            pl.BlockSpec((NX, C), lambda i: (0, i + off_blocks)),
            pl.BlockSpec((MP, C), lambda i: (0, i)),
            pl.BlockSpec((H, NX), lambda i: (0, 0)),
            pl.BlockSpec((H, 1), lambda i: (0, 0)),
            pl.BlockSpec((MP, H), lambda i: (0, 0)),
            pl.BlockSpec((MP, 1), lambda i: (0, 0)),
        ],
        out_specs=[
            pl.BlockSpec((MP, C), lambda i: (0, i)),
            pl.BlockSpec((1, C), lambda i: (0, i)),
        ],
        out_shape=[
            jax.ShapeDtypeStruct((MP, N), jnp.float32),
            jax.ShapeDtypeStruct((1, N), jnp.float32),
        ],
        compiler_params=pltpu.CompilerParams(
            dimension_semantics=("arbitrary",),
        ),
    )(xT, predsT, w1T, b1c, w2T, b2c)

    ensemble = ens.reshape(Tt, B)
    w = w8[:M].reshape(M, Tt, B).transpose(1, 2, 0)
    return ensemble, w
